# Initial kernel scaffold; baseline (speedup 1.0000x reference)
import jax
import jax.numpy as jnp
from jax.experimental import pallas as pl


def kernel(x, W1, b1, Wmu, bmu, Wlv, blv, edge_index):
    raise NotImplementedError("write your pallas kernel here")


# speedup vs baseline: 12.9919x; 12.9919x over previous
import functools

import jax
import jax.numpy as jnp
from jax import lax
from jax.experimental import pallas as pl
from jax.experimental.pallas import tpu as pltpu
from jax.experimental.pallas import tpu_sc as plsc

N_NODES = 10000
N_EDGES = 160000
N_FEAT = 256
HIDDEN = 512
N_CLASSES = 128

NC = 2
NS = 16
CHUNK = 128
N_CHUNKS = N_EDGES // CHUNK
STRIPE = 624
TAIL = N_NODES - NS * STRIPE
TAIL_OFF = NS * STRIPE

_mesh = plsc.VectorSubcoreMesh(core_axis_name="c", subcore_axis_name="s")


@functools.partial(
    pl.kernel,
    out_type=(
        jax.ShapeDtypeStruct((N_NODES, 16), jnp.float32),
        jax.ShapeDtypeStruct((N_NODES, 16), jnp.float32),
    ),
    mesh=_mesh,
    scratch_types=[
        pltpu.VMEM((CHUNK,), jnp.int32),
        pltpu.VMEM((CHUNK, 16), jnp.float32),
        pltpu.VMEM_SHARED((N_NODES, 16), jnp.float32),
    ],
)
def _deg_kernel(dst_hbm, zeros_hbm, dega_hbm, degb_hbm, didx_v, ones_v, acc):
    c = lax.axis_index("c")
    s = lax.axis_index("s")

    def fill_ones(i, _):
        ones_v[i, :] = jnp.full((16,), 1.0, dtype=jnp.float32)
        return 0

    lax.fori_loop(0, CHUNK, fill_ones, 0)

    r0 = s * STRIPE
    pltpu.sync_copy(zeros_hbm.at[pl.ds(r0, STRIPE)],
                    acc.at[pl.ds(r0, STRIPE)])

    @pl.when(s == NS - 1)
    def _():
        pltpu.sync_copy(zeros_hbm.at[pl.ds(TAIL_OFF, TAIL)],
                        acc.at[pl.ds(TAIL_OFF, TAIL)])

    plsc.subcore_barrier()

    half = N_CHUNKS // NC
    first = c * half + s
    trip = (half - s + NS - 1) // NS

    def body(j, _):
        base = (first + j * NS) * CHUNK
        pltpu.sync_copy(dst_hbm.at[pl.ds(base, CHUNK)], didx_v)
        pltpu.sync_copy(ones_v, acc.at[didx_v], add=True)
        return 0

    lax.fori_loop(0, trip, body, 0)
    plsc.subcore_barrier()

    def writeout(out_hbm):
        pltpu.sync_copy(acc.at[pl.ds(r0, STRIPE)],
                        out_hbm.at[pl.ds(r0, STRIPE)])

        @pl.when(s == NS - 1)
        def _():
            pltpu.sync_copy(acc.at[pl.ds(TAIL_OFF, TAIL)],
                            out_hbm.at[pl.ds(TAIL_OFF, TAIL)])

    @pl.when(c == 0)
    def _():
        writeout(dega_hbm)

    @pl.when(c == 1)
    def _():
        writeout(degb_hbm)


@functools.partial(
    pl.kernel,
    out_type=(
        jax.ShapeDtypeStruct((N_NODES, 128), jnp.float32),
        jax.ShapeDtypeStruct((N_NODES, 128), jnp.float32),
    ),
    mesh=_mesh,
    scratch_types=[
        pltpu.VMEM((CHUNK,), jnp.int32),
        pltpu.VMEM((CHUNK,), jnp.int32),
        pltpu.VMEM((CHUNK, 128), jnp.float32),
        pltpu.VMEM_SHARED((N_NODES, 128), jnp.float32),
        pltpu.SemaphoreType.DMA,
    ],
)
def _prop_kernel(ya_hbm, yb_hbm, src_hbm, dst_hbm, za_hbm, zb_hbm,
                 sidx_v, didx_v, rows_v, acc, gsem):
    c = lax.axis_index("c")
    s = lax.axis_index("s")
    r0 = s * STRIPE

    def run(y_hbm, out_hbm):
        pltpu.sync_copy(y_hbm.at[pl.ds(r0, STRIPE)],
                        acc.at[pl.ds(r0, STRIPE)])

        @pl.when(s == NS - 1)
        def _():
            pltpu.sync_copy(y_hbm.at[pl.ds(TAIL_OFF, TAIL)],
                            acc.at[pl.ds(TAIL_OFF, TAIL)])

        plsc.subcore_barrier()

        trip = (N_CHUNKS - s + NS - 1) // NS

        def body(j, _):
            base = (s + j * NS) * CHUNK
            pltpu.sync_copy(src_hbm.at[pl.ds(base, CHUNK)], sidx_v)
            pltpu.async_copy(y_hbm.at[sidx_v], rows_v, gsem).wait()
            pltpu.sync_copy(dst_hbm.at[pl.ds(base, CHUNK)], didx_v)
            pltpu.sync_copy(rows_v, acc.at[didx_v], add=True)
            return 0

        lax.fori_loop(0, trip, body, 0)
        plsc.subcore_barrier()
        pltpu.sync_copy(acc.at[pl.ds(r0, STRIPE)],
                        out_hbm.at[pl.ds(r0, STRIPE)])

        @pl.when(s == NS - 1)
        def _():
            pltpu.sync_copy(acc.at[pl.ds(TAIL_OFF, TAIL)],
                            out_hbm.at[pl.ds(TAIL_OFF, TAIL)])

    @pl.when(c == 0)
    def _():
        run(ya_hbm, za_hbm)

    @pl.when(c == 1)
    def _():
        run(yb_hbm, zb_hbm)


BLK = 1000
GRID = N_NODES // BLK


def _scale_body(dega, degb, x, dinv16, y0a, y0b):
    deg = dega[...] + degb[...] + 1.0
    dv16 = lax.rsqrt(deg)
    dinv16[...] = dv16
    y = x[...] * dv16[:, 0:1]
    y0a[...] = y[:, :128]
    y0b[...] = y[:, 128:]


def _mm_body(za, zb, dinv16, W1a, W1b, b1, Wmu, Wlv, y1a, y1b):
    dv = dinv16[...][:, 0:1]
    xpa = za[...] * dv
    xpb = zb[...] * dv
    h = jnp.dot(xpa, W1a[...], preferred_element_type=jnp.float32)
    h = h + jnp.dot(xpb, W1b[...], preferred_element_type=jnp.float32)
    h = jax.nn.relu(h + b1[...])
    y1a[...] = jnp.dot(h, Wmu[...], preferred_element_type=jnp.float32) * dv
    y1b[...] = jnp.dot(h, Wlv[...], preferred_element_type=jnp.float32) * dv


def _out_body(z1a, z1b, dinv16, bmu, blv, mu, lv):
    dv = dinv16[...][:, 0:1]
    mu[...] = z1a[...] * dv + bmu[...]
    lv[...] = z1b[...] * dv + blv[...]


def _row_spec(cols):
    return pl.BlockSpec((BLK, cols), lambda i: (i, 0))


def _full_spec(r, cols):
    return pl.BlockSpec((r, cols), lambda i: (0, 0))


_scale_call = pl.pallas_call(
    _scale_body,
    grid=(GRID,),
    in_specs=[_row_spec(16), _row_spec(16), _row_spec(N_FEAT)],
    out_specs=[_row_spec(16), _row_spec(128), _row_spec(128)],
    out_shape=[
        jax.ShapeDtypeStruct((N_NODES, 16), jnp.float32),
        jax.ShapeDtypeStruct((N_NODES, 128), jnp.float32),
        jax.ShapeDtypeStruct((N_NODES, 128), jnp.float32),
    ],
)

_mm_call = pl.pallas_call(
    _mm_body,
    grid=(GRID,),
    in_specs=[
        _row_spec(128), _row_spec(128), _row_spec(16),
        _full_spec(128, HIDDEN), _full_spec(128, HIDDEN), _full_spec(1, HIDDEN),
        _full_spec(HIDDEN, 128), _full_spec(HIDDEN, 128),
    ],
    out_specs=[_row_spec(128), _row_spec(128)],
    out_shape=[
        jax.ShapeDtypeStruct((N_NODES, 128), jnp.float32),
        jax.ShapeDtypeStruct((N_NODES, 128), jnp.float32),
    ],
)

_out_call = pl.pallas_call(
    _out_body,
    grid=(GRID,),
    in_specs=[
        _row_spec(128), _row_spec(128), _row_spec(16),
        _full_spec(1, 128), _full_spec(1, 128),
    ],
    out_specs=[_row_spec(128), _row_spec(128)],
    out_shape=[
        jax.ShapeDtypeStruct((N_NODES, 128), jnp.float32),
        jax.ShapeDtypeStruct((N_NODES, 128), jnp.float32),
    ],
)


def kernel(x, W1, b1, Wmu, bmu, Wlv, blv, edge_index):
    src = edge_index[0]
    dst = edge_index[1]
    zeros16 = jnp.zeros((N_NODES, 16), jnp.float32)

    dega, degb = _deg_kernel(dst, zeros16)
    dinv16, y0a, y0b = _scale_call(dega, degb, x)
    z0a, z0b = _prop_kernel(y0a, y0b, src, dst)
    y1a, y1b = _mm_call(z0a, z0b, dinv16,
                        W1[:128, :], W1[128:, :], b1.reshape(1, HIDDEN),
                        Wmu, Wlv)
    z1a, z1b = _prop_kernel(y1a, y1b, src, dst)
    mu, lv = _out_call(z1a, z1b, dinv16,
                       bmu.reshape(1, N_CLASSES), blv.reshape(1, N_CLASSES))
    return (mu, lv)


# speedup vs baseline: 24.8290x; 1.9111x over previous
import functools

import jax
import jax.numpy as jnp
from jax import lax
from jax.experimental import pallas as pl
from jax.experimental.pallas import tpu as pltpu
from jax.experimental.pallas import tpu_sc as plsc

N_NODES = 10000
N_EDGES = 160000
N_FEAT = 256
HIDDEN = 512
N_CLASSES = 128

NC = 2
NS = 16
CHUNK = 128
N_CHUNKS = N_EDGES // CHUNK
SLAB = 80
N_CHUNKS_PAD = NS * SLAB
STRIPE = 624
TAIL = N_NODES - NS * STRIPE
TAIL_OFF = NS * STRIPE

_mesh = plsc.VectorSubcoreMesh(core_axis_name="c", subcore_axis_name="s")


DEG_SLAB = N_CHUNKS_PAD // (NC * NS)


@functools.partial(
    pl.kernel,
    out_type=(
        jax.ShapeDtypeStruct((N_NODES, 16), jnp.float32),
        jax.ShapeDtypeStruct((N_NODES, 16), jnp.float32),
    ),
    mesh=_mesh,
    scratch_types=[
        pltpu.VMEM((DEG_SLAB, CHUNK), jnp.int32),
        pltpu.VMEM((CHUNK, 16), jnp.float32),
        pltpu.VMEM_SHARED((N_NODES, 16), jnp.float32),
    ],
)
def _deg_kernel(dst2_hbm, zeros_hbm, dega_hbm, degb_hbm, dslab_v, ones_v, acc):
    c = lax.axis_index("c")
    s = lax.axis_index("s")

    def fill_ones(i, _):
        ones_v[i, :] = jnp.full((16,), 1.0, dtype=jnp.float32)
        return 0

    lax.fori_loop(0, CHUNK, fill_ones, 0)

    r0 = s * STRIPE
    pltpu.sync_copy(zeros_hbm.at[pl.ds(r0, STRIPE)],
                    acc.at[pl.ds(r0, STRIPE)])

    @pl.when(s == NS - 1)
    def _():
        pltpu.sync_copy(zeros_hbm.at[pl.ds(TAIL_OFF, TAIL)],
                        acc.at[pl.ds(TAIL_OFF, TAIL)])

    off = (s * NC + c) * DEG_SLAB
    pltpu.sync_copy(dst2_hbm.at[pl.ds(off, DEG_SLAB)], dslab_v)
    plsc.subcore_barrier()

    trip = jnp.minimum(DEG_SLAB, jnp.maximum(0, N_CHUNKS - off))

    def body(j, _):
        pltpu.sync_copy(ones_v, acc.at[dslab_v.at[j]], add=True)
        return 0

    lax.fori_loop(0, trip, body, 0)
    plsc.subcore_barrier()

    def writeout(out_hbm):
        pltpu.sync_copy(acc.at[pl.ds(r0, STRIPE)],
                        out_hbm.at[pl.ds(r0, STRIPE)])

        @pl.when(s == NS - 1)
        def _():
            pltpu.sync_copy(acc.at[pl.ds(TAIL_OFF, TAIL)],
                            out_hbm.at[pl.ds(TAIL_OFF, TAIL)])

    @pl.when(c == 0)
    def _():
        writeout(dega_hbm)

    @pl.when(c == 1)
    def _():
        writeout(degb_hbm)


SLABH = SLAB // 2


@functools.partial(
    pl.kernel,
    out_type=(
        jax.ShapeDtypeStruct((N_NODES, 128), jnp.float32),
        jax.ShapeDtypeStruct((N_NODES, 128), jnp.float32),
    ),
    mesh=_mesh,
    scratch_types=[
        pltpu.VMEM((SLABH, CHUNK), jnp.int32),
        pltpu.VMEM((SLABH, CHUNK), jnp.int32),
        pltpu.VMEM((CHUNK, 128), jnp.float32),
        pltpu.VMEM((CHUNK, 128), jnp.float32),
        pltpu.VMEM_SHARED((N_NODES, 128), jnp.float32),
        pltpu.SemaphoreType.DMA,
        pltpu.SemaphoreType.DMA,
    ],
)
def _prop_kernel(ya_hbm, yb_hbm, src2_hbm, dst2_hbm, za_hbm, zb_hbm,
                 sslab_v, dslab_v, rows0_v, rows1_v, acc, gsem0, gsem1):
    c = lax.axis_index("c")
    s = lax.axis_index("s")
    r0 = s * STRIPE

    def run(y_hbm, out_hbm):
        pltpu.sync_copy(y_hbm.at[pl.ds(r0, STRIPE)],
                        acc.at[pl.ds(r0, STRIPE)])

        @pl.when(s == NS - 1)
        def _():
            pltpu.sync_copy(y_hbm.at[pl.ds(TAIL_OFF, TAIL)],
                            acc.at[pl.ds(TAIL_OFF, TAIL)])

        plsc.subcore_barrier()

        for h in range(2):
            off = s * SLAB + h * SLABH
            pltpu.sync_copy(src2_hbm.at[pl.ds(off, SLABH)], sslab_v)
            pltpu.sync_copy(dst2_hbm.at[pl.ds(off, SLABH)], dslab_v)

            trip = jnp.minimum(SLABH, N_CHUNKS - off)

            pltpu.async_copy(y_hbm.at[sslab_v.at[0]], rows0_v, gsem0)
            pltpu.async_copy(y_hbm.at[sslab_v.at[1]], rows1_v, gsem1)

            def stage(j, rows_v, gsem):
                pltpu.make_async_copy(
                    y_hbm.at[sslab_v.at[j]], rows_v, gsem).wait()
                pltpu.sync_copy(rows_v, acc.at[dslab_v.at[j]], add=True)

                @pl.when(j + 2 < trip)
                def _():
                    pltpu.async_copy(y_hbm.at[sslab_v.at[j + 2]], rows_v, gsem)

            def body(jj, _):
                stage(2 * jj, rows0_v, gsem0)
                stage(2 * jj + 1, rows1_v, gsem1)
                return 0

            lax.fori_loop(0, trip // 2, body, 0)

        plsc.subcore_barrier()
        pltpu.sync_copy(acc.at[pl.ds(r0, STRIPE)],
                        out_hbm.at[pl.ds(r0, STRIPE)])

        @pl.when(s == NS - 1)
        def _():
            pltpu.sync_copy(acc.at[pl.ds(TAIL_OFF, TAIL)],
                            out_hbm.at[pl.ds(TAIL_OFF, TAIL)])

    @pl.when(c == 0)
    def _():
        run(ya_hbm, za_hbm)

    @pl.when(c == 1)
    def _():
        run(yb_hbm, zb_hbm)


BLK = 1000
GRID = N_NODES // BLK


def _scale_body(dega, degb, x, dinv16, y0a, y0b):
    deg = dega[...] + degb[...] + 1.0
    dv16 = lax.rsqrt(deg)
    dinv16[...] = dv16
    y = x[...] * dv16[:, 0:1]
    y0a[...] = y[:, :128]
    y0b[...] = y[:, 128:]


def _mm_body(za, zb, dinv16, W1a, W1b, b1, Wmu, Wlv, y1a, y1b):
    dv = dinv16[...][:, 0:1]
    xpa = za[...] * dv
    xpb = zb[...] * dv
    h = jnp.dot(xpa, W1a[...], preferred_element_type=jnp.float32)
    h = h + jnp.dot(xpb, W1b[...], preferred_element_type=jnp.float32)
    h = jax.nn.relu(h + b1[...])
    y1a[...] = jnp.dot(h, Wmu[...], preferred_element_type=jnp.float32) * dv
    y1b[...] = jnp.dot(h, Wlv[...], preferred_element_type=jnp.float32) * dv


def _out_body(z1a, z1b, dinv16, bmu, blv, mu, lv):
    dv = dinv16[...][:, 0:1]
    mu[...] = z1a[...] * dv + bmu[...]
    lv[...] = z1b[...] * dv + blv[...]


def _row_spec(cols):
    return pl.BlockSpec((BLK, cols), lambda i: (i, 0))


def _full_spec(r, cols):
    return pl.BlockSpec((r, cols), lambda i: (0, 0))


_scale_call = pl.pallas_call(
    _scale_body,
    grid=(GRID,),
    in_specs=[_row_spec(16), _row_spec(16), _row_spec(N_FEAT)],
    out_specs=[_row_spec(16), _row_spec(128), _row_spec(128)],
    out_shape=[
        jax.ShapeDtypeStruct((N_NODES, 16), jnp.float32),
        jax.ShapeDtypeStruct((N_NODES, 128), jnp.float32),
        jax.ShapeDtypeStruct((N_NODES, 128), jnp.float32),
    ],
)

_mm_call = pl.pallas_call(
    _mm_body,
    grid=(GRID,),
    in_specs=[
        _row_spec(128), _row_spec(128), _row_spec(16),
        _full_spec(128, HIDDEN), _full_spec(128, HIDDEN), _full_spec(1, HIDDEN),
        _full_spec(HIDDEN, 128), _full_spec(HIDDEN, 128),
    ],
    out_specs=[_row_spec(128), _row_spec(128)],
    out_shape=[
        jax.ShapeDtypeStruct((N_NODES, 128), jnp.float32),
        jax.ShapeDtypeStruct((N_NODES, 128), jnp.float32),
    ],
)

_out_call = pl.pallas_call(
    _out_body,
    grid=(GRID,),
    in_specs=[
        _row_spec(128), _row_spec(128), _row_spec(16),
        _full_spec(1, 128), _full_spec(1, 128),
    ],
    out_specs=[_row_spec(128), _row_spec(128)],
    out_shape=[
        jax.ShapeDtypeStruct((N_NODES, 128), jnp.float32),
        jax.ShapeDtypeStruct((N_NODES, 128), jnp.float32),
    ],
)


def kernel(x, W1, b1, Wmu, bmu, Wlv, blv, edge_index):
    pad = N_CHUNKS_PAD - N_CHUNKS
    src2 = jnp.pad(edge_index[0].reshape(N_CHUNKS, CHUNK), ((0, pad), (0, 0)))
    dst2 = jnp.pad(edge_index[1].reshape(N_CHUNKS, CHUNK), ((0, pad), (0, 0)))
    zeros16 = jnp.zeros((N_NODES, 16), jnp.float32)

    dega, degb = _deg_kernel(dst2, zeros16)
    dinv16, y0a, y0b = _scale_call(dega, degb, x)
    z0a, z0b = _prop_kernel(y0a, y0b, src2, dst2)
    y1a, y1b = _mm_call(z0a, z0b, dinv16,
                        W1[:128, :], W1[128:, :], b1.reshape(1, HIDDEN),
                        Wmu, Wlv)
    z1a, z1b = _prop_kernel(y1a, y1b, src2, dst2)
    mu, lv = _out_call(z1a, z1b, dinv16,
                       bmu.reshape(1, N_CLASSES), blv.reshape(1, N_CLASSES))
    return (mu, lv)
